# Initial kernel scaffold; baseline (speedup 1.0000x reference)
import jax
import jax.numpy as jnp
from jax.experimental import pallas as pl


def kernel(hidden_states, Wf, bf, Wv, bv, Wo, bo, Wg, bg, W1, b1, W2, b2):
    raise NotImplementedError("write your pallas kernel here")


# speedup vs baseline: 5.5395x; 5.5395x over previous
import functools

import jax
import jax.numpy as jnp
from jax import lax
from jax.experimental import pallas as pl
from jax.experimental.pallas import tpu as pltpu
from jax.experimental.pallas import tpu_sc as plsc

B, S, D = 2, 2048, 768
E, F = 16, 1024
N = B * S

SB = 256
NSB = S // SB
TBM = 512
NTB = N // TBM
TB = 128
NB = N // TB + E
NP = NB * TB

_NC, _NS = 2, 16
NW = _NC * _NS
TPW = N // NW


def _mixer_body(x_ref, wf_ref, bf_ref, wv_ref, bv_ref, wo_ref, bo_ref,
                wg_ref, bg_ref, y_ref, sel_ref, carry):
    j = pl.program_id(1)
    x = x_ref[0]
    f = jax.nn.sigmoid(
        jnp.dot(x, wf_ref[...], preferred_element_type=jnp.float32)
        + bf_ref[...])
    v = jnp.dot(x, wv_ref[...], preferred_element_type=jnp.float32) + bv_ref[...]
    a = f
    b = (1.0 - f) * v
    k = 1
    while k < SB:
        a_sh = jnp.concatenate([jnp.ones((k, D), jnp.float32), a[:-k]], axis=0)
        b_sh = jnp.concatenate([jnp.zeros((k, D), jnp.float32), b[:-k]], axis=0)
        b = b_sh * a + b
        a = a_sh * a
        k *= 2

    @pl.when(j == 0)
    def _():
        carry[...] = jnp.zeros_like(carry)

    h = b + a * carry[...]
    carry[...] = h[SB - 1:SB, :]
    y = x + jnp.dot(h, wo_ref[...], preferred_element_type=jnp.float32) + bo_ref[...]
    y_ref[0] = y
    logits = jnp.dot(y, wg_ref[...], preferred_element_type=jnp.float32) + bg_ref[...]
    m = jnp.max(logits, axis=1, keepdims=True)
    ex = jnp.exp(logits - m)
    p = ex / jnp.sum(ex, axis=1, keepdims=True)
    pm = jnp.max(p, axis=1, keepdims=True)
    eidx = lax.broadcasted_iota(jnp.int32, (SB, E), 1)
    sel = jnp.min(jnp.where(p == pm, eidx, E), axis=1, keepdims=True)
    sel_ref[...] = sel.reshape(1, SB, 1)


def _mixer_call(x, Wf, bf, Wv, bv, Wo, bo, Wg, bg):
    full = lambda shape: pl.BlockSpec(shape, lambda i, j: (0,) * len(shape))
    return pl.pallas_call(
        _mixer_body,
        grid=(B, NSB),
        in_specs=[
            pl.BlockSpec((1, SB, D), lambda i, j: (i, j, 0)),
            full((D, D)), full((1, D)),
            full((D, D)), full((1, D)),
            full((D, D)), full((1, D)),
            full((D, E)), full((1, E)),
        ],
        out_specs=[
            pl.BlockSpec((1, SB, D), lambda i, j: (i, j, 0)),
            pl.BlockSpec((1, SB, 1), lambda i, j: (i * NSB + j, 0, 0)),
        ],
        out_shape=[
            jax.ShapeDtypeStruct((B, S, D), jnp.float32),
            jax.ShapeDtypeStruct((B * NSB, SB, 1), jnp.int32),
        ],
        scratch_shapes=[pltpu.VMEM((1, D), jnp.float32)],
    )(x, Wf, bf.reshape(1, D), Wv, bv.reshape(1, D),
      Wo, bo.reshape(1, D), Wg, bg.reshape(1, E))


def _meta_body(sel_ref, pos_ref, b2e_ref, counts, running, starts):
    ph = pl.program_id(0)
    j = pl.program_id(1)
    srow = sel_ref[0]
    ecol = lax.broadcasted_iota(jnp.int32, (E, TBM), 0)
    oh = (srow == ecol).astype(jnp.float32)

    @pl.when(ph == 0)
    def _():
        @pl.when(j == 0)
        def _():
            counts[...] = jnp.zeros_like(counts)
        counts[...] += jnp.sum(oh, axis=1, keepdims=True)

    @pl.when(ph == 1)
    def _():
        @pl.when(j == 0)
        def _():
            padded = jnp.ceil(counts[...] * (1.0 / TB)) * TB
            ii = lax.broadcasted_iota(jnp.int32, (E, E), 0)
            jj = lax.broadcasted_iota(jnp.int32, (E, E), 1)
            tril = (jj < ii).astype(jnp.float32)
            starts[...] = jnp.dot(tril, padded, preferred_element_type=jnp.float32)
            running[...] = jnp.zeros_like(running)

        ti = lax.broadcasted_iota(jnp.int32, (TBM, TBM), 0)
        tj = lax.broadcasted_iota(jnp.int32, (TBM, TBM), 1)
        triu = (ti < tj).astype(jnp.float32)
        cum = jnp.dot(oh, triu, preferred_element_type=jnp.float32)
        base = starts[...] + running[...]
        posf = jnp.sum(oh * (cum + base), axis=0, keepdims=True)
        pos_ref[...] = posf.reshape(1, 1, TBM).astype(jnp.int32)
        running[...] += jnp.sum(oh, axis=1, keepdims=True)

        padded = jnp.ceil(counts[...] * (1.0 / TB)) * TB
        endb = (starts[...] + padded) * (1.0 / TB)
        irow = lax.broadcasted_iota(jnp.int32, (E, 128), 1).astype(jnp.float32)
        b2e = jnp.sum((irow >= endb).astype(jnp.float32), axis=0, keepdims=True)
        b2e = jnp.minimum(b2e, float(E - 1))
        total = jnp.sum(padded) * (1.0 / TB)
        b2e_ref[...] = jnp.concatenate(
            [b2e, jnp.full((1, 128), total, jnp.float32),
             jnp.zeros((6, 128), jnp.float32)], axis=0)


def _meta_call(sel):
    return pl.pallas_call(
        _meta_body,
        grid=(2, NTB),
        in_specs=[pl.BlockSpec((1, 1, TBM), lambda p, j: (j, 0, 0))],
        out_specs=[
            pl.BlockSpec((1, 1, TBM), lambda p, j: (p * j, 0, 0)),
            pl.BlockSpec((8, 128), lambda p, j: (0, 0)),
        ],
        out_shape=[
            jax.ShapeDtypeStruct((NTB, 1, TBM), jnp.int32),
            jax.ShapeDtypeStruct((8, 128), jnp.float32),
        ],
        scratch_shapes=[pltpu.VMEM((E, 1), jnp.float32),
                        pltpu.VMEM((E, 1), jnp.float32),
                        pltpu.VMEM((E, 1), jnp.float32)],
    )(sel)


@functools.lru_cache(maxsize=None)
def _sc_kernels():
    mesh = plsc.VectorSubcoreMesh(core_axis_name="c", subcore_axis_name="s",
                                  num_cores=_NC, num_subcores=_NS)

    @functools.partial(
        pl.kernel, mesh=mesh,
        out_type=jax.ShapeDtypeStruct((NP, D), jnp.float32),
        scratch_types=[pltpu.VMEM((TPW,), jnp.int32),
                       pltpu.VMEM((TPW, D), jnp.float32),
                       pltpu.SemaphoreType.DMA],
    )
    def _scatter(y_hbm, pos_hbm, out_hbm, idx_v, rows_v, sem):
        wid = lax.axis_index("s") * _NC + lax.axis_index("c")
        base = wid * TPW
        pltpu.sync_copy(pos_hbm.at[pl.ds(base, TPW)], idx_v)
        pltpu.sync_copy(y_hbm.at[pl.ds(base, TPW)], rows_v)
        pltpu.async_copy(rows_v, out_hbm.at[idx_v], sem).wait()

    @functools.partial(
        pl.kernel, mesh=mesh,
        out_type=jax.ShapeDtypeStruct((N, D), jnp.float32),
        scratch_types=[pltpu.VMEM((TPW,), jnp.int32),
                       pltpu.VMEM((TPW, D), jnp.float32),
                       pltpu.SemaphoreType.DMA],
    )
    def _gather(src_hbm, pos_hbm, out_hbm, idx_v, rows_v, sem):
        wid = lax.axis_index("s") * _NC + lax.axis_index("c")
        base = wid * TPW
        pltpu.sync_copy(pos_hbm.at[pl.ds(base, TPW)], idx_v)
        pltpu.async_copy(src_hbm.at[idx_v], rows_v, sem).wait()
        pltpu.sync_copy(rows_v, out_hbm.at[pl.ds(base, TPW)])

    return _scatter, _gather


def _ffn_body(b2e_ref, nb_ref, x_ref, w1_ref, b1_ref, w2_ref, b2_ref, o_ref):
    i = pl.program_id(0)

    @pl.when(i < nb_ref[0])
    def _():
        x = x_ref[...]
        h = jnp.dot(x, w1_ref[0], preferred_element_type=jnp.float32) + b1_ref[0]
        h = jax.nn.gelu(h)
        o_ref[...] = (jnp.dot(h, w2_ref[0], preferred_element_type=jnp.float32)
                      + b2_ref[0])


def _ffn_call(b2e, nbu, xs, W1, b1, W2, b2):
    grid_spec = pltpu.PrefetchScalarGridSpec(
        num_scalar_prefetch=2,
        grid=(NB,),
        in_specs=[
            pl.BlockSpec((TB, D), lambda i, m, n: (i, 0)),
            pl.BlockSpec((1, D, F), lambda i, m, n: (m[i], 0, 0)),
            pl.BlockSpec((1, 1, F), lambda i, m, n: (m[i], 0, 0)),
            pl.BlockSpec((1, F, D), lambda i, m, n: (m[i], 0, 0)),
            pl.BlockSpec((1, 1, D), lambda i, m, n: (m[i], 0, 0)),
        ],
        out_specs=pl.BlockSpec((TB, D), lambda i, m, n: (i, 0)),
    )
    return pl.pallas_call(
        _ffn_body,
        grid_spec=grid_spec,
        out_shape=jax.ShapeDtypeStruct((NP, D), jnp.float32),
    )(b2e, nbu, xs, W1, b1.reshape(E, 1, F), W2, b2.reshape(E, 1, D))


def kernel(hidden_states, Wf, bf, Wv, bv, Wo, bo, Wg, bg, W1, b1, W2, b2):
    y, sel3 = _mixer_call(hidden_states, Wf, bf, Wv, bv, Wo, bo, Wg, bg)
    sel = sel3.reshape(NTB, 1, TBM)
    posr, meta = _meta_call(sel)
    pos = posr.reshape(N)
    b2e = meta[0, :NB].astype(jnp.int32)
    nbu = meta[1, :1].astype(jnp.int32)
    sc_scatter, sc_gather = _sc_kernels()
    ysorted = sc_scatter(y.reshape(N, D), pos)
    osorted = _ffn_call(b2e, nbu, ysorted, W1, b1, W2, b2)
    final = sc_gather(osorted, pos)
    return final.reshape(B, S, D)


# speedup vs baseline: 5.5785x; 1.0070x over previous
import functools

import jax
import jax.numpy as jnp
from jax import lax
from jax.experimental import pallas as pl
from jax.experimental.pallas import tpu as pltpu
from jax.experimental.pallas import tpu_sc as plsc

B, S, D = 2, 2048, 768
E, F = 16, 1024
N = B * S

SB = 256
NSB = S // SB
G = B * NSB
TB = 128
NB = N // TB + E
NP = NB * TB

_NC, _NS = 2, 16
NW = _NC * _NS
TPW = N // NW


def _mixer_body(x_ref, wf_ref, bf_ref, wv_ref, bv_ref, wo_ref, bo_ref,
                wg_ref, bg_ref, y_ref, pos_ref, m2_ref,
                carry, sel_s, lr_s, pc_s):
    g = pl.program_id(0)

    @pl.when(g < G)
    def _mix():
        x = x_ref[0]
        f = jax.nn.sigmoid(
            jnp.dot(x, wf_ref[...], preferred_element_type=jnp.float32)
            + bf_ref[...])
        v = (jnp.dot(x, wv_ref[...], preferred_element_type=jnp.float32)
             + bv_ref[...])
        a = f
        b = (1.0 - f) * v
        k = 1
        while k < SB:
            a_sh = jnp.concatenate(
                [jnp.ones((k, D), jnp.float32), a[:-k]], axis=0)
            b_sh = jnp.concatenate(
                [jnp.zeros((k, D), jnp.float32), b[:-k]], axis=0)
            b = b_sh * a + b
            a = a_sh * a
            k *= 2

        @pl.when(g % NSB == 0)
        def _():
            carry[...] = jnp.zeros_like(carry)

        h = b + a * carry[...]
        carry[...] = h[SB - 1:SB, :]
        y = (x + jnp.dot(h, wo_ref[...], preferred_element_type=jnp.float32)
             + bo_ref[...])
        y_ref[0] = y

        logits = (jnp.dot(y, wg_ref[...], preferred_element_type=jnp.float32)
                  + bg_ref[...])
        m = jnp.max(logits, axis=1, keepdims=True)
        ex = jnp.exp(logits - m)
        p = ex / jnp.sum(ex, axis=1, keepdims=True)
        pm = jnp.max(p, axis=1, keepdims=True)
        eidx = lax.broadcasted_iota(jnp.int32, (SB, E), 1)
        sel = jnp.min(jnp.where(p == pm, eidx, E), axis=1, keepdims=True)

        oh = (sel == eidx).astype(jnp.float32)
        ti = lax.broadcasted_iota(jnp.int32, (SB, SB), 0)
        tj = lax.broadcasted_iota(jnp.int32, (SB, SB), 1)
        trist = (tj < ti).astype(jnp.float32)
        cum = jnp.dot(trist, oh, preferred_element_type=jnp.float32)
        lrank = jnp.sum(oh * cum, axis=1, keepdims=True)
        base = pl.multiple_of(g * SB, SB)
        sel_s[pl.ds(base, SB)] = sel
        lr_s[pl.ds(base, SB)] = lrank
        pc_s[pl.ds(g, 1), :] = jnp.sum(oh, axis=0, keepdims=True)

    @pl.when(g == G)
    def _meta():
        pc = pc_s[...]
        counts = jnp.sum(pc, axis=0, keepdims=True)
        padded = jnp.ceil(counts * (1.0 / TB)) * TB
        i16 = lax.broadcasted_iota(jnp.int32, (E, E), 0)
        j16 = lax.broadcasted_iota(jnp.int32, (E, E), 1)
        excl = (i16 < j16).astype(jnp.float32)
        starts = jnp.dot(padded, excl, preferred_element_type=jnp.float32)
        gi = lax.broadcasted_iota(jnp.int32, (G, G), 0)
        gj = lax.broadcasted_iota(jnp.int32, (G, G), 1)
        gtri = (gj < gi).astype(jnp.float32)
        offs = (jnp.dot(gtri, pc, preferred_element_type=jnp.float32)
                + starts)
        erow = lax.broadcasted_iota(jnp.int32, (SB, E), 1)
        for gg in range(G):
            sc = sel_s[gg * SB:(gg + 1) * SB]
            ohg = (sc == erow).astype(jnp.float32)
            pos_g = (jnp.sum(ohg * offs[gg:gg + 1, :], axis=1, keepdims=True)
                     + lr_s[gg * SB:(gg + 1) * SB])
            pos_ref[gg * SB:(gg + 1) * SB] = pos_g.astype(jnp.int32)
        endb = (starts + padded) * (1.0 / TB)
        icol = lax.broadcasted_iota(jnp.int32, (128, E), 0).astype(jnp.float32)
        b2e = jnp.sum((icol >= endb).astype(jnp.float32), axis=1,
                      keepdims=True)
        b2e = jnp.minimum(b2e, float(E - 1))
        total = jnp.sum(padded) * (1.0 / TB)
        m2_ref[...] = jnp.concatenate(
            [b2e, jnp.full((128, 1), total, jnp.float32)], axis=0)


def _mixer_call(x, Wf, bf, Wv, bv, Wo, bo, Wg, bg):
    full = lambda shape: pl.BlockSpec(shape, lambda g: (0,) * len(shape))

    def xmap(g):
        gc = jnp.minimum(g, G - 1)
        return (gc // NSB, gc % NSB, 0)

    return pl.pallas_call(
        _mixer_body,
        grid=(G + 1,),
        in_specs=[
            pl.BlockSpec((1, SB, D), xmap),
            full((D, D)), full((1, D)),
            full((D, D)), full((1, D)),
            full((D, D)), full((1, D)),
            full((D, E)), full((1, E)),
        ],
        out_specs=[
            pl.BlockSpec((1, SB, D), xmap),
            pl.BlockSpec((N, 1), lambda g: (0, 0)),
            pl.BlockSpec((256, 1), lambda g: (0, 0)),
        ],
        out_shape=[
            jax.ShapeDtypeStruct((B, S, D), jnp.float32),
            jax.ShapeDtypeStruct((N, 1), jnp.int32),
            jax.ShapeDtypeStruct((256, 1), jnp.float32),
        ],
        scratch_shapes=[pltpu.VMEM((1, D), jnp.float32),
                        pltpu.VMEM((N, 1), jnp.int32),
                        pltpu.VMEM((N, 1), jnp.float32),
                        pltpu.VMEM((G, E), jnp.float32)],
    )(x, Wf, bf.reshape(1, D), Wv, bv.reshape(1, D),
      Wo, bo.reshape(1, D), Wg, bg.reshape(1, E))


@functools.lru_cache(maxsize=None)
def _sc_kernels():
    mesh = plsc.VectorSubcoreMesh(core_axis_name="c", subcore_axis_name="s",
                                  num_cores=_NC, num_subcores=_NS)

    @functools.partial(
        pl.kernel, mesh=mesh,
        out_type=jax.ShapeDtypeStruct((NP, D), jnp.float32),
        scratch_types=[pltpu.VMEM((TPW,), jnp.int32),
                       pltpu.VMEM((TPW, D), jnp.float32),
                       pltpu.SemaphoreType.DMA],
    )
    def _scatter(y_hbm, pos_hbm, out_hbm, idx_v, rows_v, sem):
        wid = lax.axis_index("s") * _NC + lax.axis_index("c")
        base = wid * TPW
        pltpu.sync_copy(pos_hbm.at[pl.ds(base, TPW)], idx_v)
        pltpu.sync_copy(y_hbm.at[pl.ds(base, TPW)], rows_v)
        pltpu.async_copy(rows_v, out_hbm.at[idx_v], sem).wait()

    @functools.partial(
        pl.kernel, mesh=mesh,
        out_type=jax.ShapeDtypeStruct((N, D), jnp.float32),
        scratch_types=[pltpu.VMEM((TPW,), jnp.int32),
                       pltpu.VMEM((TPW, D), jnp.float32),
                       pltpu.SemaphoreType.DMA],
    )
    def _gather(src_hbm, pos_hbm, out_hbm, idx_v, rows_v, sem):
        wid = lax.axis_index("s") * _NC + lax.axis_index("c")
        base = wid * TPW
        pltpu.sync_copy(pos_hbm.at[pl.ds(base, TPW)], idx_v)
        pltpu.async_copy(src_hbm.at[idx_v], rows_v, sem).wait()
        pltpu.sync_copy(rows_v, out_hbm.at[pl.ds(base, TPW)])

    return _scatter, _gather


def _ffn_body(b2e_ref, nb_ref, x_ref, w1_ref, b1_ref, w2_ref, b2_ref, o_ref):
    i = pl.program_id(0)

    @pl.when(i < nb_ref[0])
    def _():
        x = x_ref[...]
        h = jnp.dot(x, w1_ref[0], preferred_element_type=jnp.float32) + b1_ref[0]
        h = jax.nn.gelu(h)
        o_ref[...] = (jnp.dot(h, w2_ref[0], preferred_element_type=jnp.float32)
                      + b2_ref[0])


def _ffn_call(b2e, nbu, xs, W1, b1, W2, b2):
    grid_spec = pltpu.PrefetchScalarGridSpec(
        num_scalar_prefetch=2,
        grid=(NB,),
        in_specs=[
            pl.BlockSpec((TB, D), lambda i, m, n: (i, 0)),
            pl.BlockSpec((1, D, F), lambda i, m, n: (m[i], 0, 0)),
            pl.BlockSpec((1, 1, F), lambda i, m, n: (m[i], 0, 0)),
            pl.BlockSpec((1, F, D), lambda i, m, n: (m[i], 0, 0)),
            pl.BlockSpec((1, 1, D), lambda i, m, n: (m[i], 0, 0)),
        ],
        out_specs=pl.BlockSpec((TB, D), lambda i, m, n: (i, 0)),
    )
    return pl.pallas_call(
        _ffn_body,
        grid_spec=grid_spec,
        out_shape=jax.ShapeDtypeStruct((NP, D), jnp.float32),
    )(b2e, nbu, xs, W1, b1.reshape(E, 1, F), W2, b2.reshape(E, 1, D))


def kernel(hidden_states, Wf, bf, Wv, bv, Wo, bo, Wg, bg, W1, b1, W2, b2):
    y, posc, m2 = _mixer_call(hidden_states, Wf, bf, Wv, bv, Wo, bo, Wg, bg)
    pos = posc.reshape(N)
    b2e = m2[:NB, 0].astype(jnp.int32)
    nbu = m2[128:129, 0].astype(jnp.int32)
    sc_scatter, sc_gather = _sc_kernels()
    ysorted = sc_scatter(y.reshape(N, D), pos)
    osorted = _ffn_call(b2e, nbu, ysorted, W1, b1, W2, b2)
    final = sc_gather(osorted, pos)
    return final.reshape(B, S, D)
